# Initial kernel scaffold; baseline (speedup 1.0000x reference)
import jax
import jax.numpy as jnp
from jax.experimental import pallas as pl


def kernel(node_features, sup_masses, actions, angles, gt_edges, round_n, params, edge_index, sub_graphs_0, sep_subgraphs_0):
    raise NotImplementedError("write your pallas kernel here")


# speedup vs baseline: 2.5940x; 2.5940x over previous
import functools

import jax
import jax.numpy as jnp
from jax import lax
from jax.experimental import pallas as pl
from jax.experimental.pallas import tpu as pltpu
from jax.experimental.pallas import tpu_sc as plsc

D = 128
N = 10000
E = 160000
E2 = 2 * E
MS = 131072
MAX_EP_LEN = 50.0

NCH2 = 2560
CPW2 = NCH2 // 16
E2PAD = NCH2 * 128
NCH1 = 1280
CPW1 = NCH1 // 16
E1PAD = NCH1 * 128
AGG_ROWS = 10240
NCHS = MS // 128
CPWS = NCHS // 16
TROWS = 4 * N + 8
SEG_R = 8192
NPASS = MS // SEG_R
EPW5 = MS // 16

_f32 = jnp.float32
_i32 = jnp.int32


def _lrelu(x):
    return jnp.maximum(x, 0.01 * x)


def _tc1_body(x_ref, w_ref, b_ref, o_ref):
    o_ref[...] = jnp.dot(x_ref[...], w_ref[...],
                         preferred_element_type=_f32) + b_ref[...]


def _tc1(x, w, b):
    return pl.pallas_call(
        _tc1_body,
        grid=(10,),
        in_specs=[
            pl.BlockSpec((1000, D), lambda i: (i, 0)),
            pl.BlockSpec((D, 512), lambda i: (0, 0)),
            pl.BlockSpec((1, 512), lambda i: (0, 0)),
        ],
        out_specs=pl.BlockSpec((1000, 512), lambda i: (i, 0)),
        out_shape=jax.ShapeDtypeStruct((N, 512), _f32),
    )(x, w, b)


def _tc2_body(x_ref, agg_ref, u1_ref, u2_ref, bu_ref, we_ref, be_ref, o_ref):
    x = x_ref[...]
    outs = []
    for q in range(2):
        h = _lrelu(jnp.dot(x, u1_ref[q], preferred_element_type=_f32)
                   + jnp.dot(agg_ref[q], u2_ref[q], preferred_element_type=_f32)
                   + bu_ref[q])
        outs.append(jnp.dot(h, we_ref[q], preferred_element_type=_f32) + be_ref[q])
    o_ref[...] = jnp.concatenate(outs, axis=1)


def _tc2(x, agg, u1, u2, bu, we, be):
    return pl.pallas_call(
        _tc2_body,
        grid=(10,),
        in_specs=[
            pl.BlockSpec((1000, D), lambda i: (i, 0)),
            pl.BlockSpec((2, 1000, D), lambda i: (0, i, 0)),
            pl.BlockSpec((2, D, D), lambda i: (0, 0, 0)),
            pl.BlockSpec((2, D, D), lambda i: (0, 0, 0)),
            pl.BlockSpec((2, 1, D), lambda i: (0, 0, 0)),
            pl.BlockSpec((2, D, 2 * D), lambda i: (0, 0, 0)),
            pl.BlockSpec((2, 1, 2 * D), lambda i: (0, 0, 0)),
        ],
        out_specs=pl.BlockSpec((1000, 512), lambda i: (i, 0)),
        out_shape=jax.ShapeDtypeStruct((N, 512), _f32),
    )(x, agg, u1, u2, bu, we, be)


def _tc3_body(sub_ref, w_ref, b_ref, o_ref):
    o_ref[...] = jnp.dot(sub_ref[...], w_ref[0],
                         preferred_element_type=_f32) + b_ref[0]


def _tc3(sub, w, b):
    return pl.pallas_call(
        _tc3_body,
        grid=(2, 64),
        in_specs=[
            pl.BlockSpec((2048, D), lambda q, i: (q * 64 + i, 0)),
            pl.BlockSpec((1, D, 384), lambda q, i: (q, 0, 0)),
            pl.BlockSpec((1, 1, 384), lambda q, i: (q, 0, 0)),
        ],
        out_specs=pl.BlockSpec((2048, 384), lambda q, i: (q * 64 + i, 0)),
        out_shape=jax.ShapeDtypeStruct((2 * MS, 384), _f32),
    )(sub, w, b)


def _tc4_body(g_ref, agg2_ref, w_ref, z_ref, zsq_ref):
    sub_u1 = g_ref[...][:, 256:384]
    mm = jnp.dot(agg2_ref[0], w_ref[0], preferred_element_type=_f32)
    h = _lrelu(sub_u1 + mm)
    hsq = h * h
    for g in range(16):
        z_ref[0, g, :] = jnp.mean(h[g * 128:(g + 1) * 128], axis=0)
        zsq_ref[0, g, :] = jnp.mean(hsq[g * 128:(g + 1) * 128], axis=0)


def _tc4(ginter, agg2, w):
    return pl.pallas_call(
        _tc4_body,
        grid=(2, 64),
        in_specs=[
            pl.BlockSpec((2048, 384), lambda q, i: (q * 64 + i, 0)),
            pl.BlockSpec((1, 2048, D), lambda q, i: (q, i, 0)),
            pl.BlockSpec((1, D, D), lambda q, i: (q, 0, 0)),
        ],
        out_specs=[
            pl.BlockSpec((1, 16, D), lambda q, i: (q, i, 0)),
            pl.BlockSpec((1, 16, D), lambda q, i: (q, i, 0)),
        ],
        out_shape=[
            jax.ShapeDtypeStruct((2, 1024, D), _f32),
            jax.ShapeDtypeStruct((2, 1024, D), _f32),
        ],
    )(ginter, agg2, w)


def _tc5_body(z_ref, zsq_ref, bn0_ref, w1_ref, a1_ref, w2_ref, a2_ref,
              w3_ref, b3_ref, q_ref, gsl_ref):
    z = z_ref[0]
    gsl = jnp.mean(zsq_ref[0])

    def bn_lrelu(a, g, b):
        mu = jnp.mean(a, axis=0)
        var = jnp.mean(a * a, axis=0) - mu * mu
        return _lrelu((a - mu) * lax.rsqrt(var + 1e-5) * g + b)

    z0 = bn_lrelu(z, bn0_ref[0, 0], bn0_ref[0, 1])
    a1 = jnp.dot(z0, w1_ref[0], preferred_element_type=_f32) + a1_ref[0, 0]
    a1 = bn_lrelu(a1, a1_ref[0, 1], a1_ref[0, 2])
    a2 = jnp.dot(a1, w2_ref[0], preferred_element_type=_f32) + a2_ref[0, 0]
    a2 = bn_lrelu(a2, a2_ref[0, 1], a2_ref[0, 2])
    q_ref[0] = jnp.dot(a2, w3_ref[0], preferred_element_type=_f32) + b3_ref[0]
    gsl_ref[0] = jnp.full((8, D), gsl, _f32)


def _tc5(z, zsq, bn0, w1, a1, w2, a2, w3, b3):
    return pl.pallas_call(
        _tc5_body,
        grid=(2,),
        in_specs=[
            pl.BlockSpec((1, 1024, D), lambda q: (q, 0, 0)),
            pl.BlockSpec((1, 1024, D), lambda q: (q, 0, 0)),
            pl.BlockSpec((1, 2, D), lambda q: (q, 0, 0)),
            pl.BlockSpec((1, D, 256), lambda q: (q, 0, 0)),
            pl.BlockSpec((1, 3, 256), lambda q: (q, 0, 0)),
            pl.BlockSpec((1, 256, 256), lambda q: (q, 0, 0)),
            pl.BlockSpec((1, 3, 256), lambda q: (q, 0, 0)),
            pl.BlockSpec((1, 256, D), lambda q: (q, 0, 0)),
            pl.BlockSpec((1, 1, D), lambda q: (q, 0, 0)),
        ],
        out_specs=[
            pl.BlockSpec((1, 1024, D), lambda q: (q, 0, 0)),
            pl.BlockSpec((1, 8, D), lambda q: (q, 0, 0)),
        ],
        out_shape=[
            jax.ShapeDtypeStruct((2, 1024, D), _f32),
            jax.ShapeDtypeStruct((2, 8, D), _f32),
        ],
    )(z, zsq, bn0, w1, a1, w2, a2, w3, b3)


def _sc_mesh():
    return plsc.VectorSubcoreMesh(core_axis_name="c", subcore_axis_name="s")


def _sc_msg_call(t_tab, srcidx, dstidx, scatidx, zer):
    @functools.partial(
        pl.kernel,
        mesh=_sc_mesh(),
        out_type=jax.ShapeDtypeStruct((2, AGG_ROWS, D), _f32),
        scratch_types=[
            pltpu.VMEM((8, 128), _i32),
            pltpu.VMEM((8, 128), _i32),
            pltpu.VMEM((8, 128), _i32),
            pltpu.VMEM((128, D), _f32),
            pltpu.VMEM((128, D), _f32),
            pltpu.VMEM_SHARED((AGG_ROWS, D), _f32),
            pltpu.SemaphoreType.DMA,
            pltpu.SemaphoreType.DMA,
        ],
    )
    def body(t_hbm, src_hbm, dst_hbm, sc_hbm, zer_hbm, out_hbm,
             sidx, didx, scat, xsr, xdr, aggsh, sem1, sem2):
        c = lax.axis_index("c")
        s = lax.axis_index("s")
        for t in range(5):
            pltpu.sync_copy(zer_hbm.at[pl.ds(0, 128)],
                            aggsh.at[pl.ds(s * 640 + t * 128, 128)])
        plsc.subcore_barrier()

        def group(gg, _):
            base = s * CPW2 + gg * 8
            pltpu.sync_copy(src_hbm.at[c, pl.ds(base, 8)], sidx)
            pltpu.sync_copy(dst_hbm.at[c, pl.ds(base, 8)], didx)
            pltpu.sync_copy(sc_hbm.at[pl.ds(base, 8)], scat)

            def chunk(j, _):
                cp1 = pltpu.async_copy(t_hbm.at[sidx.at[j]], xsr, sem1)
                cp2 = pltpu.async_copy(t_hbm.at[didx.at[j]], xdr, sem2)
                cp1.wait()
                cp2.wait()

                def row(r, _):
                    for b in range(8):
                        sl = pl.ds(b * 16, 16)
                        t = xsr[r, sl] + xdr[r, sl]
                        xsr[r, sl] = jnp.maximum(t, 0.01 * t)
                    return 0

                lax.fori_loop(0, 128, row, 0)
                pltpu.sync_copy(xsr, aggsh.at[scat.at[j]], add=True)
                return 0

            lax.fori_loop(0, 8, chunk, 0)
            return 0

        lax.fori_loop(0, CPW2 // 8, group, 0)
        plsc.subcore_barrier()
        pltpu.sync_copy(aggsh.at[pl.ds(s * 640, 640)],
                        out_hbm.at[c, pl.ds(s * 640, 640)])

    return body(t_tab, srcidx, dstidx, scatidx, zer)


def _sc_edge_call(h_tab, srcidx, dstidx, ang, act, gt, wv, subidx):
    @functools.partial(
        pl.kernel,
        mesh=_sc_mesh(),
        out_type=(
            jax.ShapeDtypeStruct((2 * E1PAD, D), _f32),
            jax.ShapeDtypeStruct((2, 128, 16), _f32),
            jax.ShapeDtypeStruct((2 * MS, D), _f32),
        ),
        scratch_types=[
            pltpu.VMEM((CPW1, 128), _i32),
            pltpu.VMEM((CPW1, 128), _i32),
            pltpu.VMEM((CPW1, 128), _f32),
            pltpu.VMEM((CPW1, 128), _f32),
            pltpu.VMEM((CPW1, 128), _f32),
            pltpu.VMEM((3, D), _f32),
            pltpu.VMEM((CPWS, 128), _i32),
            pltpu.VMEM((128, D), _f32),
            pltpu.VMEM((128, D), _f32),
            pltpu.VMEM((8, 16), _f32),
            pltpu.SemaphoreType.DMA,
            pltpu.SemaphoreType.DMA,
        ],
    )
    def body(h_hbm, src_hbm, dst_hbm, ang_hbm, act_hbm, gt_hbm, wv_hbm,
             sub_hbm, ef_hbm, sse_hbm, subo_hbm,
             sidx, didx, angv, actv, gtv, wvv, subi, hsr, hdr, ssev,
             sem1, sem2):
        c = lax.axis_index("c")
        s = lax.axis_index("s")
        pltpu.sync_copy(src_hbm.at[c, pl.ds(s * CPW1, CPW1)], sidx)
        pltpu.sync_copy(dst_hbm.at[c, pl.ds(s * CPW1, CPW1)], didx)
        pltpu.sync_copy(ang_hbm.at[pl.ds(s * CPW1, CPW1)], angv)
        pltpu.sync_copy(act_hbm.at[pl.ds(s * CPW1, CPW1)], actv)
        pltpu.sync_copy(gt_hbm.at[pl.ds(s * CPW1, CPW1)], gtv)
        pltpu.sync_copy(wv_hbm.at[c], wvv)
        pltpu.sync_copy(sub_hbm.at[c, pl.ds(s * CPWS, CPWS)], subi)

        wa = [wvv[0, pl.ds(b * 16, 16)] for b in range(8)]
        wb = [wvv[1, pl.ds(b * 16, 16)] for b in range(8)]
        ws = [wvv[2, pl.ds(b * 16, 16)] for b in range(8)]
        lanes = lax.iota(_i32, 16)
        perms = [lanes ^ k for k in (8, 4, 2, 1)]
        efbase = (c * NCH1 + s * CPW1) * 128

        def chunk(j, sse):
            cp1 = pltpu.async_copy(h_hbm.at[sidx.at[j]], hsr, sem1)
            cp2 = pltpu.async_copy(h_hbm.at[didx.at[j]], hdr, sem2)
            cp1.wait()
            cp2.wait()

            def grp(g, sse_in):
                svec = jnp.zeros((16,), _f32)
                av = angv[j, pl.ds(g * 16, 16)]
                cv = actv[j, pl.ds(g * 16, 16)]
                for r16 in range(16):
                    r = g * 16 + r16
                    a_s = av[r16]
                    c_s = cv[r16]
                    p = None
                    for b in range(8):
                        sl = pl.ds(b * 16, 16)
                        t = hsr[r, sl] + hdr[r, sl] + a_s * wa[b] + c_s * wb[b]
                        e = jnp.maximum(t, 0.01 * t)
                        hsr[r, sl] = e
                        p = e * ws[b] if p is None else p + e * ws[b]
                    for pm in perms:
                        p = p + p.at[pm].get(mode="promise_in_bounds")
                    svec = jnp.where(lanes == r16, p, svec)
                sig = 1.0 / (1.0 + jnp.exp(-svec))
                dlt = sig - gtv[j, pl.ds(g * 16, 16)]
                return sse_in + dlt * dlt

            sse = lax.fori_loop(0, 8, grp, sse)
            pltpu.sync_copy(hsr, ef_hbm.at[pl.ds(efbase + j * 128, 128)])
            return sse

        sse = lax.fori_loop(0, CPW1, chunk, jnp.zeros((16,), _f32))
        ssev[0, :] = sse
        for t in range(1, 8):
            ssev[t, :] = jnp.zeros((16,), _f32)
        pltpu.sync_copy(ssev, sse_hbm.at[c, pl.ds(s * 8, 8)])
        plsc.subcore_barrier()

        subbase = (c * NCHS + s * CPWS) * 128

        def sg(k, _):
            pltpu.async_copy(ef_hbm.at[subi.at[k]], hsr, sem1).wait()
            pltpu.sync_copy(hsr, subo_hbm.at[pl.ds(subbase + k * 128, 128)])
            return 0

        lax.fori_loop(0, CPWS, sg, 0)

    return body(h_tab, srcidx, dstidx, ang, act, gt, wv, subidx)


WCAP = 8320


def _sc_sub_call(g_tab, gs, gd, zer):
    @functools.partial(
        pl.kernel,
        mesh=_sc_mesh(),
        out_type=jax.ShapeDtypeStruct((2, MS, D), _f32),
        scratch_types=[
            pltpu.VMEM((EPW5,), _i32),
            pltpu.VMEM((EPW5,), _i32),
            pltpu.VMEM((1, 128), _i32),
            pltpu.VMEM((128,), _i32),
            pltpu.VMEM((128,), _i32),
            pltpu.VMEM((128,), _i32),
            pltpu.VMEM((64,), _i32),
            pltpu.VMEM((64,), _i32),
            pltpu.VMEM((64,), _i32),
            pltpu.VMEM((64, D), _f32),
            pltpu.VMEM((64, D), _f32),
            pltpu.VMEM((48,), _i32),
            pltpu.VMEM_SHARED((16 * WCAP,), _i32),
            pltpu.VMEM_SHARED((16 * WCAP,), _i32),
            pltpu.VMEM_SHARED((16 * WCAP,), _i32),
            pltpu.VMEM_SHARED((SEG_R + 128, D), _f32),
            pltpu.SemaphoreType.DMA,
            pltpu.SemaphoreType.DMA,
        ],
    )
    def body(g_hbm, gs_hbm, gd_hbm, zer_hbm, out_hbm,
             gsv, gdv, tgtstg, vals, vald, valr, cidxs, cidxd, cidxr,
             arows, brows, cntv, bkts, bktd, bktr, spm, sem1, sem2):
        c = lax.axis_index("c")
        s = lax.axis_index("s")
        cq = c * (3 * MS)
        wbase = s * WCAP
        trash = wbase + WCAP - 1
        lanes = lax.iota(_i32, 16)
        pltpu.sync_copy(gs_hbm.at[pl.ds(s * EPW5, EPW5)], gsv)
        pltpu.sync_copy(gd_hbm.at[pl.ds(s * EPW5, EPW5)], gdv)
        for t in range(4):
            pltpu.sync_copy(zer_hbm.at[pl.ds(0, 128)],
                            spm.at[pl.ds(s * 520 + t * 128, 128)])
        pltpu.sync_copy(zer_hbm.at[pl.ds(0, 8)],
                        spm.at[pl.ds(s * 520 + 512, 8)])
        plsc.subcore_barrier()

        kmask = {k: jnp.minimum(jnp.maximum(lanes - (k - 1), 0), 1)
                 for k in (1, 2, 4, 8)}

        def pass_body(p, _):
            lo = p * SEG_R

            def batch(bb, off_in):
                off_b = off_in
                for gg in range(8):
                    sl = pl.ds(bb * 128 + gg * 16, 16)
                    gsvv = gsv[sl]
                    gdvv = gdv[sl]
                    rel = gdvv - lo
                    ind = ((rel >> 31) + 1) * ((rel - SEG_R) >> 31) * (-1)
                    pf = ind
                    for k in (1, 2, 4, 8):
                        sh = pf.at[jnp.maximum(lanes - k, 0)].get(
                            mode="promise_in_bounds")
                        pf = pf + sh * kmask[k]
                    osl = pl.ds(gg * 16, 16)
                    tgtstg[0, osl] = trash + (wbase + off_b + pf - 1
                                              - trash) * ind
                    vals[osl] = 3 * gsvv + cq
                    vald[osl] = 3 * gdvv + (cq + 1)
                    valr[osl] = rel
                    off_b = off_b + pf[15]
                pltpu.sync_copy(vals, bkts.at[tgtstg.at[0]])
                pltpu.sync_copy(vald, bktd.at[tgtstg.at[0]])
                pltpu.sync_copy(valr, bktr.at[tgtstg.at[0]])
                return off_b

            off = lax.fori_loop(0, EPW5 // 128, batch, jnp.int32(0))

            for k in range(4):
                osl = pl.ds(k * 16, 16)
                tgtstg[0, osl] = wbase + off + k * 16 + lanes
                vals[osl] = jnp.zeros((16,), _i32)
                vald[osl] = jnp.zeros((16,), _i32)
                valr[osl] = jnp.full((16,), SEG_R, _i32)
            for k in range(4, 8):
                tgtstg[0, pl.ds(k * 16, 16)] = jnp.zeros((16,), _i32) + trash
            pltpu.sync_copy(vals, bkts.at[tgtstg.at[0]])
            pltpu.sync_copy(vald, bktd.at[tgtstg.at[0]])
            pltpu.sync_copy(valr, bktr.at[tgtstg.at[0]])
            nch = (off + 63) // 64

            def surv(j, _):
                cb = pl.multiple_of(wbase + j * 64, 64)
                pltpu.sync_copy(bkts.at[pl.ds(cb, 64)], cidxs)
                pltpu.sync_copy(bktd.at[pl.ds(cb, 64)], cidxd)
                pltpu.sync_copy(bktr.at[pl.ds(cb, 64)], cidxr)
                cp1 = pltpu.async_copy(g_hbm.at[cidxs], arows, sem1)
                cp2 = pltpu.async_copy(g_hbm.at[cidxd], brows, sem2)
                cp1.wait()
                cp2.wait()

                def row(r, _):
                    for b in range(8):
                        sl = pl.ds(b * 16, 16)
                        t = arows[r, sl] + brows[r, sl]
                        arows[r, sl] = jnp.maximum(t, 0.01 * t)
                    return 0

                lax.fori_loop(0, 64, row, 0)
                for t16 in range(4):
                    relv = cidxr[pl.ds(t16 * 16, 16)]
                    pltpu.sync_copy(arows.at[pl.ds(t16 * 16, 16)],
                                    spm.at[relv], add=True)
                return 0

            lax.fori_loop(0, nch, surv, 0)
            plsc.subcore_barrier()
            pltpu.sync_copy(spm.at[pl.ds(s * 512, 512)],
                            out_hbm.at[c, pl.ds(lo + s * 512, 512)])
            for t in range(4):
                pltpu.sync_copy(zer_hbm.at[pl.ds(0, 128)],
                                spm.at[pl.ds(s * 512 + t * 128, 128)])

            @pl.when(s == 15)
            def _():
                pltpu.sync_copy(zer_hbm.at[pl.ds(0, 128)],
                                spm.at[pl.ds(SEG_R, 128)])

            plsc.subcore_barrier()
            return 0

        lax.fori_loop(0, NPASS, pass_body, 0)

    return body(g_tab, gs, gd, zer)


def _pad_i32(a, n, val):
    return jnp.concatenate([a.astype(_i32), jnp.full((n - a.shape[0],), val, _i32)])


def _pad_f32(a, n, val):
    return jnp.concatenate([a.astype(_f32), jnp.full((n - a.shape[0],), val, _f32)])


def kernel(node_features, sup_masses, actions, angles, gt_edges, round_n,
           params, edge_index, sub_graphs_0, sep_subgraphs_0):
    p1, p2 = params["q1"], params["q2"]
    g1, g2 = params["g1"], params["g2"]
    v1, v2 = params["v1"], params["v2"]

    rn = round_n / MAX_EP_LEN
    x = jnp.concatenate(
        [node_features, sup_masses, jnp.ones_like(sup_masses) * rn], axis=1)

    s0 = edge_index[0].astype(_i32)
    d0 = edge_index[1].astype(_i32)
    e2src = jnp.concatenate([s0, d0])
    e2dst = jnp.concatenate([d0, s0])

    srcabs2 = jnp.stack([_pad_i32(4 * e2src + 2 * q, E2PAD, 4 * N)
                         for q in range(2)]).reshape(2, NCH2, 128)
    dstabs2 = jnp.stack([_pad_i32(4 * e2dst + 2 * q + 1, E2PAD, 4 * N)
                         for q in range(2)]).reshape(2, NCH2, 128)
    scat2 = _pad_i32(e2dst, E2PAD, N).reshape(NCH2, 128)

    srcabs1 = jnp.stack([_pad_i32(4 * s0 + 2 * q, E1PAD, 4 * N)
                         for q in range(2)]).reshape(2, NCH1, 128)
    dstabs1 = jnp.stack([_pad_i32(4 * d0 + 2 * q + 1, E1PAD, 4 * N)
                         for q in range(2)]).reshape(2, NCH1, 128)
    angp = _pad_f32(angles, E1PAD, 0.0).reshape(NCH1, 128)
    actp = _pad_f32(actions, E1PAD, 0.0).reshape(NCH1, 128)
    gtp = _pad_f32(gt_edges, E1PAD, 0.5).reshape(NCH1, 128)

    subidx = jnp.stack([sub_graphs_0.astype(_i32) + q * E1PAD
                        for q in range(2)]).reshape(2, NCHS, 128)

    gs = jnp.concatenate([sep_subgraphs_0[0], sep_subgraphs_0[1]]).astype(_i32)
    gd = jnp.concatenate([sep_subgraphs_0[1], sep_subgraphs_0[0]]).astype(_i32)

    zer = jnp.zeros((128, D), _f32)

    wcat = jnp.concatenate([p1["W_msg"][:D], p1["W_msg"][D:],
                            p2["W_msg"][:D], p2["W_msg"][D:]], axis=1)
    bcat = jnp.concatenate([jnp.zeros((D,), _f32), p1["b_msg"],
                            jnp.zeros((D,), _f32), p2["b_msg"]])[None, :]
    t_tab = jnp.concatenate([_tc1(x, wcat, bcat).reshape(4 * N, D),
                             jnp.zeros((8, D), _f32)], axis=0)

    agg = _sc_msg_call(t_tab, srcabs2, dstabs2, scat2, zer)[:, :N]

    u1 = jnp.stack([p1["W_upd"][:D], p2["W_upd"][:D]])
    u2 = jnp.stack([p1["W_upd"][D:], p2["W_upd"][D:]])
    bu = jnp.stack([p1["b_upd"], p2["b_upd"]])[:, None, :]
    we = jnp.stack([jnp.concatenate([p1["W_edge"][:D], p1["W_edge"][D:2 * D]], axis=1),
                    jnp.concatenate([p2["W_edge"][:D], p2["W_edge"][D:2 * D]], axis=1)])
    be = jnp.stack([jnp.concatenate([jnp.zeros((D,), _f32), p1["b_edge"]]),
                    jnp.concatenate([jnp.zeros((D,), _f32), p2["b_edge"]])])[:, None, :]
    h_tab = jnp.concatenate([_tc2(x, agg, u1, u2, bu, we, be).reshape(4 * N, D),
                             jnp.zeros((8, D), _f32)], axis=0)

    wv = jnp.stack([jnp.stack([p1["W_edge"][2 * D], p1["W_edge"][2 * D + 1], p1["w_side"]]),
                    jnp.stack([p2["W_edge"][2 * D], p2["W_edge"][2 * D + 1], p2["w_side"]])])
    ef, sse_out, sub = _sc_edge_call(h_tab, srcabs1, dstabs1, angp, actp, gtp,
                                     wv, subidx)

    wg = jnp.stack([jnp.concatenate([g1["W_msg"][:D], g1["W_msg"][D:], g1["W_upd"][:D]], axis=1),
                    jnp.concatenate([g2["W_msg"][:D], g2["W_msg"][D:], g2["W_upd"][:D]], axis=1)])
    bg = jnp.stack([jnp.concatenate([jnp.zeros((D,), _f32), g1["b_msg"], g1["b_upd"]]),
                    jnp.concatenate([jnp.zeros((D,), _f32), g2["b_msg"], g2["b_upd"]])])[:, None, :]
    ginter = _tc3(sub, wg, bg)
    g_tab = ginter.reshape(6 * MS, D)

    agg2 = _sc_sub_call(g_tab, gs, gd, zer)

    wu2 = jnp.stack([g1["W_upd"][D:], g2["W_upd"][D:]])
    z, zsq = _tc4(ginter, agg2, wu2)

    bn0 = jnp.stack([jnp.stack([v1["bn0_g"], v1["bn0_b"]]),
                     jnp.stack([v2["bn0_g"], v2["bn0_b"]])])
    w1 = jnp.stack([v1["W1"], v2["W1"]])
    a1 = jnp.stack([jnp.stack([v1["b1"], v1["bn1_g"], v1["bn1_b"]]),
                    jnp.stack([v2["b1"], v2["bn1_g"], v2["bn1_b"]])])
    w2 = jnp.stack([v1["W2"], v2["W2"]])
    a2 = jnp.stack([jnp.stack([v1["b2"], v1["bn2_g"], v1["bn2_b"]]),
                    jnp.stack([v2["b2"], v2["bn2_g"], v2["bn2_b"]])])
    w3 = jnp.stack([jnp.pad(v1["W3"], ((0, 0), (0, D - 1))),
                    jnp.pad(v2["W3"], ((0, 0), (0, D - 1)))])
    b3 = jnp.stack([jnp.broadcast_to(v1["b3"], (1, D)),
                    jnp.broadcast_to(v2["b3"], (1, D))])
    qf, gslf = _tc5(z, zsq, bn0, w1, a1, w2, a2, w3, b3)

    q1o = qf[0, :, 0]
    q2o = qf[1, :, 0]
    side = (jnp.sum(sse_out[0]) / E + jnp.sum(sse_out[1]) / E
            + gslf[0, 0, 0] + gslf[1, 0, 0]) / 4.0
    return (q1o, q2o, side)


# speedup vs baseline: 2.6153x; 1.0082x over previous
import functools

import jax
import jax.numpy as jnp
from jax import lax
from jax.experimental import pallas as pl
from jax.experimental.pallas import tpu as pltpu
from jax.experimental.pallas import tpu_sc as plsc

D = 128
N = 10000
E = 160000
E2 = 2 * E
MS = 131072
MAX_EP_LEN = 50.0

NCH2 = 2560
CPW2 = NCH2 // 16
E2PAD = NCH2 * 128
NCH1 = 1280
CPW1 = NCH1 // 16
E1PAD = NCH1 * 128
AGG_ROWS = 10240
NCHS = MS // 128
CPWS = NCHS // 16
TROWS = 4 * N + 8
SEG_R = 8192
NPASS = MS // SEG_R
EPW5 = MS // 16

_f32 = jnp.float32
_i32 = jnp.int32


def _lrelu(x):
    return jnp.maximum(x, 0.01 * x)


def _tc1_body(x_ref, w_ref, b_ref, o_ref):
    o_ref[...] = jnp.dot(x_ref[...], w_ref[...],
                         preferred_element_type=_f32) + b_ref[...]


def _tc1(x, w, b):
    return pl.pallas_call(
        _tc1_body,
        grid=(10,),
        in_specs=[
            pl.BlockSpec((1000, D), lambda i: (i, 0)),
            pl.BlockSpec((D, 512), lambda i: (0, 0)),
            pl.BlockSpec((1, 512), lambda i: (0, 0)),
        ],
        out_specs=pl.BlockSpec((1000, 512), lambda i: (i, 0)),
        out_shape=jax.ShapeDtypeStruct((N, 512), _f32),
    )(x, w, b)


def _tc2_body(x_ref, agg_ref, u1_ref, u2_ref, bu_ref, we_ref, be_ref, o_ref):
    x = x_ref[...]
    outs = []
    for q in range(2):
        h = _lrelu(jnp.dot(x, u1_ref[q], preferred_element_type=_f32)
                   + jnp.dot(agg_ref[q], u2_ref[q], preferred_element_type=_f32)
                   + bu_ref[q])
        outs.append(jnp.dot(h, we_ref[q], preferred_element_type=_f32) + be_ref[q])
    o_ref[...] = jnp.concatenate(outs, axis=1)


def _tc2(x, agg, u1, u2, bu, we, be):
    return pl.pallas_call(
        _tc2_body,
        grid=(10,),
        in_specs=[
            pl.BlockSpec((1000, D), lambda i: (i, 0)),
            pl.BlockSpec((2, 1000, D), lambda i: (0, i, 0)),
            pl.BlockSpec((2, D, D), lambda i: (0, 0, 0)),
            pl.BlockSpec((2, D, D), lambda i: (0, 0, 0)),
            pl.BlockSpec((2, 1, D), lambda i: (0, 0, 0)),
            pl.BlockSpec((2, D, 2 * D), lambda i: (0, 0, 0)),
            pl.BlockSpec((2, 1, 2 * D), lambda i: (0, 0, 0)),
        ],
        out_specs=pl.BlockSpec((1000, 512), lambda i: (i, 0)),
        out_shape=jax.ShapeDtypeStruct((N, 512), _f32),
    )(x, agg, u1, u2, bu, we, be)


def _tc3_body(sub_ref, w_ref, b_ref, o_ref):
    o_ref[...] = jnp.dot(sub_ref[...], w_ref[0],
                         preferred_element_type=_f32) + b_ref[0]


def _tc3(sub, w, b):
    return pl.pallas_call(
        _tc3_body,
        grid=(2, 64),
        in_specs=[
            pl.BlockSpec((2048, D), lambda q, i: (q * 64 + i, 0)),
            pl.BlockSpec((1, D, 384), lambda q, i: (q, 0, 0)),
            pl.BlockSpec((1, 1, 384), lambda q, i: (q, 0, 0)),
        ],
        out_specs=pl.BlockSpec((2048, 384), lambda q, i: (q * 64 + i, 0)),
        out_shape=jax.ShapeDtypeStruct((2 * MS, 384), _f32),
    )(sub, w, b)


def _tc4_body(g_ref, agg2_ref, w_ref, z_ref, zsq_ref):
    sub_u1 = g_ref[...][:, 256:384]
    mm = jnp.dot(agg2_ref[0], w_ref[0], preferred_element_type=_f32)
    h = _lrelu(sub_u1 + mm)
    hsq = h * h
    for g in range(16):
        z_ref[0, g, :] = jnp.mean(h[g * 128:(g + 1) * 128], axis=0)
        zsq_ref[0, g, :] = jnp.mean(hsq[g * 128:(g + 1) * 128], axis=0)


def _tc4(ginter, agg2, w):
    return pl.pallas_call(
        _tc4_body,
        grid=(2, 64),
        in_specs=[
            pl.BlockSpec((2048, 384), lambda q, i: (q * 64 + i, 0)),
            pl.BlockSpec((1, 2048, D), lambda q, i: (q, i, 0)),
            pl.BlockSpec((1, D, D), lambda q, i: (q, 0, 0)),
        ],
        out_specs=[
            pl.BlockSpec((1, 16, D), lambda q, i: (q, i, 0)),
            pl.BlockSpec((1, 16, D), lambda q, i: (q, i, 0)),
        ],
        out_shape=[
            jax.ShapeDtypeStruct((2, 1024, D), _f32),
            jax.ShapeDtypeStruct((2, 1024, D), _f32),
        ],
    )(ginter, agg2, w)


def _tc5_body(z_ref, zsq_ref, bn0_ref, w1_ref, a1_ref, w2_ref, a2_ref,
              w3_ref, b3_ref, q_ref, gsl_ref):
    z = z_ref[0]
    gsl = jnp.mean(zsq_ref[0])

    def bn_lrelu(a, g, b):
        mu = jnp.mean(a, axis=0)
        var = jnp.mean(a * a, axis=0) - mu * mu
        return _lrelu((a - mu) * lax.rsqrt(var + 1e-5) * g + b)

    z0 = bn_lrelu(z, bn0_ref[0, 0], bn0_ref[0, 1])
    a1 = jnp.dot(z0, w1_ref[0], preferred_element_type=_f32) + a1_ref[0, 0]
    a1 = bn_lrelu(a1, a1_ref[0, 1], a1_ref[0, 2])
    a2 = jnp.dot(a1, w2_ref[0], preferred_element_type=_f32) + a2_ref[0, 0]
    a2 = bn_lrelu(a2, a2_ref[0, 1], a2_ref[0, 2])
    q_ref[0] = jnp.dot(a2, w3_ref[0], preferred_element_type=_f32) + b3_ref[0]
    gsl_ref[0] = jnp.full((8, D), gsl, _f32)


def _tc5(z, zsq, bn0, w1, a1, w2, a2, w3, b3):
    return pl.pallas_call(
        _tc5_body,
        grid=(2,),
        in_specs=[
            pl.BlockSpec((1, 1024, D), lambda q: (q, 0, 0)),
            pl.BlockSpec((1, 1024, D), lambda q: (q, 0, 0)),
            pl.BlockSpec((1, 2, D), lambda q: (q, 0, 0)),
            pl.BlockSpec((1, D, 256), lambda q: (q, 0, 0)),
            pl.BlockSpec((1, 3, 256), lambda q: (q, 0, 0)),
            pl.BlockSpec((1, 256, 256), lambda q: (q, 0, 0)),
            pl.BlockSpec((1, 3, 256), lambda q: (q, 0, 0)),
            pl.BlockSpec((1, 256, D), lambda q: (q, 0, 0)),
            pl.BlockSpec((1, 1, D), lambda q: (q, 0, 0)),
        ],
        out_specs=[
            pl.BlockSpec((1, 1024, D), lambda q: (q, 0, 0)),
            pl.BlockSpec((1, 8, D), lambda q: (q, 0, 0)),
        ],
        out_shape=[
            jax.ShapeDtypeStruct((2, 1024, D), _f32),
            jax.ShapeDtypeStruct((2, 8, D), _f32),
        ],
    )(z, zsq, bn0, w1, a1, w2, a2, w3, b3)


def _sc_mesh():
    return plsc.VectorSubcoreMesh(core_axis_name="c", subcore_axis_name="s")


CH2 = 64
GRP2R = 16


def _sc_msg_call(t_tab, srcidx, dstidx, scatidx, zer):
    @functools.partial(
        pl.kernel,
        mesh=_sc_mesh(),
        out_type=jax.ShapeDtypeStruct((2, AGG_ROWS, D), _f32),
        scratch_types=[
            pltpu.VMEM((GRP2R, 128), _i32),
            pltpu.VMEM((GRP2R, 128), _i32),
            pltpu.VMEM((GRP2R, 128), _i32),
            pltpu.VMEM((CH2,), _i32),
            pltpu.VMEM((CH2, D), _f32),
            pltpu.VMEM((CH2, D), _f32),
            pltpu.VMEM((CH2, D), _f32),
            pltpu.VMEM((CH2, D), _f32),
            pltpu.SemaphoreType.DMA,
            pltpu.SemaphoreType.DMA,
            pltpu.VMEM_SHARED((AGG_ROWS, D), _f32),
        ],
    )
    def body(t_hbm, src_hbm, dst_hbm, sc_hbm, zer_hbm, out_hbm,
             sidx, didx, scat, scbuf, xs0, xd0, xs1, xd1, sem0, sem1, aggsh):
        c = lax.axis_index("c")
        s = lax.axis_index("s")
        for t in range(5):
            pltpu.sync_copy(zer_hbm.at[pl.ds(0, 128)],
                            aggsh.at[pl.ds(s * 640 + t * 128, 128)])
        plsc.subcore_barrier()

        def issue(r, h, xsb, xdb, sem):
            pltpu.async_copy(t_hbm.at[sidx.at[r, pl.ds(h, CH2)]], xsb, sem)
            pltpu.async_copy(t_hbm.at[didx.at[r, pl.ds(h, CH2)]], xdb, sem)

        def wait2(xsb, xdb, sem):
            pltpu.make_async_copy(t_hbm.at[sidx.at[0, pl.ds(0, CH2)]], xsb,
                                  sem).wait()
            pltpu.make_async_copy(t_hbm.at[didx.at[0, pl.ds(0, CH2)]], xdb,
                                  sem).wait()

        def process(r, h, xsb, xdb):
            def row(rr, _):
                for u in range(4):
                    for b in range(8):
                        sl = pl.ds(b * 16, 16)
                        t = xsb[rr * 4 + u, sl] + xdb[rr * 4 + u, sl]
                        xsb[rr * 4 + u, sl] = jnp.maximum(t, 0.01 * t)
                return 0

            lax.fori_loop(0, CH2 // 4, row, 0)
            for k in range(CH2 // 16):
                scbuf[pl.ds(k * 16, 16)] = scat[r, pl.ds(h + k * 16, 16)]
            pltpu.sync_copy(xsb, aggsh.at[scbuf], add=True)

        def group(gg, _):
            base = s * (GRP2R * 10) + gg * GRP2R
            pltpu.sync_copy(src_hbm.at[c, pl.ds(base, GRP2R)], sidx)
            pltpu.sync_copy(dst_hbm.at[c, pl.ds(base, GRP2R)], didx)
            pltpu.sync_copy(sc_hbm.at[pl.ds(base, GRP2R)], scat)
            issue(0, 0, xs0, xd0, sem0)
            issue(0, CH2, xs1, xd1, sem1)

            def pair(t, _):
                tn = (t + 1) & (GRP2R - 1)
                wait2(xs0, xd0, sem0)
                process(t, 0, xs0, xd0)
                issue(tn, 0, xs0, xd0, sem0)
                wait2(xs1, xd1, sem1)
                process(t, CH2, xs1, xd1)
                issue(tn, CH2, xs1, xd1, sem1)
                return 0

            lax.fori_loop(0, GRP2R, pair, 0)
            wait2(xs0, xd0, sem0)
            wait2(xs1, xd1, sem1)
            return 0

        lax.fori_loop(0, 10, group, 0)
        plsc.subcore_barrier()
        pltpu.sync_copy(aggsh.at[pl.ds(s * 640, 640)],
                        out_hbm.at[c, pl.ds(s * 640, 640)])

    return body(t_tab, srcidx, dstidx, scatidx, zer)


def _sc_edge_call(h_tab, srcidx, dstidx, ang, act, gt, wv, subidx):
    @functools.partial(
        pl.kernel,
        mesh=_sc_mesh(),
        out_type=(
            jax.ShapeDtypeStruct((2 * E1PAD, D), _f32),
            jax.ShapeDtypeStruct((2, 128, 16), _f32),
            jax.ShapeDtypeStruct((2 * MS, D), _f32),
        ),
        scratch_types=[
            pltpu.VMEM((CPW1, 128), _i32),
            pltpu.VMEM((CPW1, 128), _i32),
            pltpu.VMEM((CPW1, 128), _f32),
            pltpu.VMEM((CPW1, 128), _f32),
            pltpu.VMEM((CPW1, 128), _f32),
            pltpu.VMEM((3, D), _f32),
            pltpu.VMEM((CPWS, 128), _i32),
            pltpu.VMEM((128, D), _f32),
            pltpu.VMEM((128, D), _f32),
            pltpu.VMEM((8, 16), _f32),
            pltpu.SemaphoreType.DMA,
            pltpu.SemaphoreType.DMA,
        ],
    )
    def body(h_hbm, src_hbm, dst_hbm, ang_hbm, act_hbm, gt_hbm, wv_hbm,
             sub_hbm, ef_hbm, sse_hbm, subo_hbm,
             sidx, didx, angv, actv, gtv, wvv, subi, hsr, hdr, ssev,
             sem1, sem2):
        c = lax.axis_index("c")
        s = lax.axis_index("s")
        pltpu.sync_copy(src_hbm.at[c, pl.ds(s * CPW1, CPW1)], sidx)
        pltpu.sync_copy(dst_hbm.at[c, pl.ds(s * CPW1, CPW1)], didx)
        pltpu.sync_copy(ang_hbm.at[pl.ds(s * CPW1, CPW1)], angv)
        pltpu.sync_copy(act_hbm.at[pl.ds(s * CPW1, CPW1)], actv)
        pltpu.sync_copy(gt_hbm.at[pl.ds(s * CPW1, CPW1)], gtv)
        pltpu.sync_copy(wv_hbm.at[c], wvv)
        pltpu.sync_copy(sub_hbm.at[c, pl.ds(s * CPWS, CPWS)], subi)

        wa = [wvv[0, pl.ds(b * 16, 16)] for b in range(8)]
        wb = [wvv[1, pl.ds(b * 16, 16)] for b in range(8)]
        ws = [wvv[2, pl.ds(b * 16, 16)] for b in range(8)]
        lanes = lax.iota(_i32, 16)
        perms = [lanes ^ k for k in (8, 4, 2, 1)]
        efbase = (c * NCH1 + s * CPW1) * 128

        def chunk(j, sse):
            cp1 = pltpu.async_copy(h_hbm.at[sidx.at[j]], hsr, sem1)
            cp2 = pltpu.async_copy(h_hbm.at[didx.at[j]], hdr, sem2)
            cp1.wait()
            cp2.wait()

            def grp(g, sse_in):
                svec = jnp.zeros((16,), _f32)
                av = angv[j, pl.ds(g * 16, 16)]
                cv = actv[j, pl.ds(g * 16, 16)]
                for r16 in range(16):
                    r = g * 16 + r16
                    a_s = av[r16]
                    c_s = cv[r16]
                    p = None
                    for b in range(8):
                        sl = pl.ds(b * 16, 16)
                        t = hsr[r, sl] + hdr[r, sl] + a_s * wa[b] + c_s * wb[b]
                        e = jnp.maximum(t, 0.01 * t)
                        hsr[r, sl] = e
                        p = e * ws[b] if p is None else p + e * ws[b]
                    for pm in perms:
                        p = p + p.at[pm].get(mode="promise_in_bounds")
                    svec = jnp.where(lanes == r16, p, svec)
                sig = 1.0 / (1.0 + jnp.exp(-svec))
                dlt = sig - gtv[j, pl.ds(g * 16, 16)]
                return sse_in + dlt * dlt

            sse = lax.fori_loop(0, 8, grp, sse)
            pltpu.sync_copy(hsr, ef_hbm.at[pl.ds(efbase + j * 128, 128)])
            return sse

        sse = lax.fori_loop(0, CPW1, chunk, jnp.zeros((16,), _f32))
        ssev[0, :] = sse
        for t in range(1, 8):
            ssev[t, :] = jnp.zeros((16,), _f32)
        pltpu.sync_copy(ssev, sse_hbm.at[c, pl.ds(s * 8, 8)])
        plsc.subcore_barrier()

        subbase = (c * NCHS + s * CPWS) * 128

        def sg(k, _):
            pltpu.async_copy(ef_hbm.at[subi.at[k]], hsr, sem1).wait()
            pltpu.sync_copy(hsr, subo_hbm.at[pl.ds(subbase + k * 128, 128)])
            return 0

        lax.fori_loop(0, CPWS, sg, 0)

    return body(h_tab, srcidx, dstidx, ang, act, gt, wv, subidx)


WCAP = 8320


def _sc_sub_call(g_tab, gs, gd, zer):
    @functools.partial(
        pl.kernel,
        mesh=_sc_mesh(),
        out_type=jax.ShapeDtypeStruct((2, MS, D), _f32),
        scratch_types=[
            pltpu.VMEM((EPW5,), _i32),
            pltpu.VMEM((EPW5,), _i32),
            pltpu.VMEM((1, 128), _i32),
            pltpu.VMEM((128,), _i32),
            pltpu.VMEM((128,), _i32),
            pltpu.VMEM((128,), _i32),
            pltpu.VMEM((64,), _i32),
            pltpu.VMEM((64,), _i32),
            pltpu.VMEM((64,), _i32),
            pltpu.VMEM((64, D), _f32),
            pltpu.VMEM((64, D), _f32),
            pltpu.VMEM((48,), _i32),
            pltpu.VMEM_SHARED((16 * WCAP,), _i32),
            pltpu.VMEM_SHARED((16 * WCAP,), _i32),
            pltpu.VMEM_SHARED((16 * WCAP,), _i32),
            pltpu.VMEM_SHARED((SEG_R + 128, D), _f32),
            pltpu.SemaphoreType.DMA,
            pltpu.SemaphoreType.DMA,
        ],
    )
    def body(g_hbm, gs_hbm, gd_hbm, zer_hbm, out_hbm,
             gsv, gdv, tgtstg, vals, vald, valr, cidxs, cidxd, cidxr,
             arows, brows, cntv, bkts, bktd, bktr, spm, sem1, sem2):
        c = lax.axis_index("c")
        s = lax.axis_index("s")
        cq = c * (3 * MS)
        wbase = s * WCAP
        trash = wbase + WCAP - 1
        lanes = lax.iota(_i32, 16)
        pltpu.sync_copy(gs_hbm.at[pl.ds(s * EPW5, EPW5)], gsv)
        pltpu.sync_copy(gd_hbm.at[pl.ds(s * EPW5, EPW5)], gdv)
        for t in range(4):
            pltpu.sync_copy(zer_hbm.at[pl.ds(0, 128)],
                            spm.at[pl.ds(s * 520 + t * 128, 128)])
        pltpu.sync_copy(zer_hbm.at[pl.ds(0, 8)],
                        spm.at[pl.ds(s * 520 + 512, 8)])
        plsc.subcore_barrier()

        kmask = {k: jnp.minimum(jnp.maximum(lanes - (k - 1), 0), 1)
                 for k in (1, 2, 4, 8)}

        def pass_body(p, _):
            lo = p * SEG_R

            def batch(bb, off_in):
                off_b = off_in
                for gg in range(8):
                    sl = pl.ds(bb * 128 + gg * 16, 16)
                    gsvv = gsv[sl]
                    gdvv = gdv[sl]
                    rel = gdvv - lo
                    ind = ((rel >> 31) + 1) * ((rel - SEG_R) >> 31) * (-1)
                    pf = ind
                    for k in (1, 2, 4, 8):
                        sh = pf.at[jnp.maximum(lanes - k, 0)].get(
                            mode="promise_in_bounds")
                        pf = pf + sh * kmask[k]
                    osl = pl.ds(gg * 16, 16)
                    tgtstg[0, osl] = trash + (wbase + off_b + pf - 1
                                              - trash) * ind
                    vals[osl] = 3 * gsvv + cq
                    vald[osl] = 3 * gdvv + (cq + 1)
                    valr[osl] = rel
                    off_b = off_b + pf[15]
                pltpu.sync_copy(vals, bkts.at[tgtstg.at[0]])
                pltpu.sync_copy(vald, bktd.at[tgtstg.at[0]])
                pltpu.sync_copy(valr, bktr.at[tgtstg.at[0]])
                return off_b

            off = lax.fori_loop(0, EPW5 // 128, batch, jnp.int32(0))

            for k in range(4):
                osl = pl.ds(k * 16, 16)
                tgtstg[0, osl] = wbase + off + k * 16 + lanes
                vals[osl] = jnp.zeros((16,), _i32)
                vald[osl] = jnp.zeros((16,), _i32)
                valr[osl] = jnp.full((16,), SEG_R, _i32)
            for k in range(4, 8):
                tgtstg[0, pl.ds(k * 16, 16)] = jnp.zeros((16,), _i32) + trash
            pltpu.sync_copy(vals, bkts.at[tgtstg.at[0]])
            pltpu.sync_copy(vald, bktd.at[tgtstg.at[0]])
            pltpu.sync_copy(valr, bktr.at[tgtstg.at[0]])
            nch = (off + 63) // 64

            def surv(j, _):
                cb = pl.multiple_of(wbase + j * 64, 64)
                pltpu.sync_copy(bkts.at[pl.ds(cb, 64)], cidxs)
                pltpu.sync_copy(bktd.at[pl.ds(cb, 64)], cidxd)
                pltpu.sync_copy(bktr.at[pl.ds(cb, 64)], cidxr)
                cp1 = pltpu.async_copy(g_hbm.at[cidxs], arows, sem1)
                cp2 = pltpu.async_copy(g_hbm.at[cidxd], brows, sem2)
                cp1.wait()
                cp2.wait()

                def row(r, _):
                    for b in range(8):
                        sl = pl.ds(b * 16, 16)
                        t = arows[r, sl] + brows[r, sl]
                        arows[r, sl] = jnp.maximum(t, 0.01 * t)
                    return 0

                lax.fori_loop(0, 64, row, 0)
                for t16 in range(4):
                    relv = cidxr[pl.ds(t16 * 16, 16)]
                    pltpu.sync_copy(arows.at[pl.ds(t16 * 16, 16)],
                                    spm.at[relv], add=True)
                return 0

            lax.fori_loop(0, nch, surv, 0)
            plsc.subcore_barrier()
            pltpu.sync_copy(spm.at[pl.ds(s * 512, 512)],
                            out_hbm.at[c, pl.ds(lo + s * 512, 512)])
            for t in range(4):
                pltpu.sync_copy(zer_hbm.at[pl.ds(0, 128)],
                                spm.at[pl.ds(s * 512 + t * 128, 128)])

            @pl.when(s == 15)
            def _():
                pltpu.sync_copy(zer_hbm.at[pl.ds(0, 128)],
                                spm.at[pl.ds(SEG_R, 128)])

            plsc.subcore_barrier()
            return 0

        lax.fori_loop(0, NPASS, pass_body, 0)

    return body(g_tab, gs, gd, zer)


def _pad_i32(a, n, val):
    return jnp.concatenate([a.astype(_i32), jnp.full((n - a.shape[0],), val, _i32)])


def _pad_f32(a, n, val):
    return jnp.concatenate([a.astype(_f32), jnp.full((n - a.shape[0],), val, _f32)])


def kernel(node_features, sup_masses, actions, angles, gt_edges, round_n,
           params, edge_index, sub_graphs_0, sep_subgraphs_0):
    p1, p2 = params["q1"], params["q2"]
    g1, g2 = params["g1"], params["g2"]
    v1, v2 = params["v1"], params["v2"]

    rn = round_n / MAX_EP_LEN
    x = jnp.concatenate(
        [node_features, sup_masses, jnp.ones_like(sup_masses) * rn], axis=1)

    s0 = edge_index[0].astype(_i32)
    d0 = edge_index[1].astype(_i32)
    e2src = jnp.concatenate([s0, d0])
    e2dst = jnp.concatenate([d0, s0])

    srcabs2 = jnp.stack([_pad_i32(4 * e2src + 2 * q, E2PAD, 4 * N)
                         for q in range(2)]).reshape(2, NCH2, 128)
    dstabs2 = jnp.stack([_pad_i32(4 * e2dst + 2 * q + 1, E2PAD, 4 * N)
                         for q in range(2)]).reshape(2, NCH2, 128)
    scat2 = _pad_i32(e2dst, E2PAD, N).reshape(NCH2, 128)

    srcabs1 = jnp.stack([_pad_i32(4 * s0 + 2 * q, E1PAD, 4 * N)
                         for q in range(2)]).reshape(2, NCH1, 128)
    dstabs1 = jnp.stack([_pad_i32(4 * d0 + 2 * q + 1, E1PAD, 4 * N)
                         for q in range(2)]).reshape(2, NCH1, 128)
    angp = _pad_f32(angles, E1PAD, 0.0).reshape(NCH1, 128)
    actp = _pad_f32(actions, E1PAD, 0.0).reshape(NCH1, 128)
    gtp = _pad_f32(gt_edges, E1PAD, 0.5).reshape(NCH1, 128)

    subidx = jnp.stack([sub_graphs_0.astype(_i32) + q * E1PAD
                        for q in range(2)]).reshape(2, NCHS, 128)

    gs = jnp.concatenate([sep_subgraphs_0[0], sep_subgraphs_0[1]]).astype(_i32)
    gd = jnp.concatenate([sep_subgraphs_0[1], sep_subgraphs_0[0]]).astype(_i32)

    zer = jnp.zeros((128, D), _f32)

    wcat = jnp.concatenate([p1["W_msg"][:D], p1["W_msg"][D:],
                            p2["W_msg"][:D], p2["W_msg"][D:]], axis=1)
    bcat = jnp.concatenate([jnp.zeros((D,), _f32), p1["b_msg"],
                            jnp.zeros((D,), _f32), p2["b_msg"]])[None, :]
    t_tab = jnp.concatenate([_tc1(x, wcat, bcat).reshape(4 * N, D),
                             jnp.zeros((8, D), _f32)], axis=0)

    agg = _sc_msg_call(t_tab, srcabs2, dstabs2, scat2, zer)[:, :N]

    u1 = jnp.stack([p1["W_upd"][:D], p2["W_upd"][:D]])
    u2 = jnp.stack([p1["W_upd"][D:], p2["W_upd"][D:]])
    bu = jnp.stack([p1["b_upd"], p2["b_upd"]])[:, None, :]
    we = jnp.stack([jnp.concatenate([p1["W_edge"][:D], p1["W_edge"][D:2 * D]], axis=1),
                    jnp.concatenate([p2["W_edge"][:D], p2["W_edge"][D:2 * D]], axis=1)])
    be = jnp.stack([jnp.concatenate([jnp.zeros((D,), _f32), p1["b_edge"]]),
                    jnp.concatenate([jnp.zeros((D,), _f32), p2["b_edge"]])])[:, None, :]
    h_tab = jnp.concatenate([_tc2(x, agg, u1, u2, bu, we, be).reshape(4 * N, D),
                             jnp.zeros((8, D), _f32)], axis=0)

    wv = jnp.stack([jnp.stack([p1["W_edge"][2 * D], p1["W_edge"][2 * D + 1], p1["w_side"]]),
                    jnp.stack([p2["W_edge"][2 * D], p2["W_edge"][2 * D + 1], p2["w_side"]])])
    ef, sse_out, sub = _sc_edge_call(h_tab, srcabs1, dstabs1, angp, actp, gtp,
                                     wv, subidx)

    wg = jnp.stack([jnp.concatenate([g1["W_msg"][:D], g1["W_msg"][D:], g1["W_upd"][:D]], axis=1),
                    jnp.concatenate([g2["W_msg"][:D], g2["W_msg"][D:], g2["W_upd"][:D]], axis=1)])
    bg = jnp.stack([jnp.concatenate([jnp.zeros((D,), _f32), g1["b_msg"], g1["b_upd"]]),
                    jnp.concatenate([jnp.zeros((D,), _f32), g2["b_msg"], g2["b_upd"]])])[:, None, :]
    ginter = _tc3(sub, wg, bg)
    g_tab = ginter.reshape(6 * MS, D)

    agg2 = _sc_sub_call(g_tab, gs, gd, zer)

    wu2 = jnp.stack([g1["W_upd"][D:], g2["W_upd"][D:]])
    z, zsq = _tc4(ginter, agg2, wu2)

    bn0 = jnp.stack([jnp.stack([v1["bn0_g"], v1["bn0_b"]]),
                     jnp.stack([v2["bn0_g"], v2["bn0_b"]])])
    w1 = jnp.stack([v1["W1"], v2["W1"]])
    a1 = jnp.stack([jnp.stack([v1["b1"], v1["bn1_g"], v1["bn1_b"]]),
                    jnp.stack([v2["b1"], v2["bn1_g"], v2["bn1_b"]])])
    w2 = jnp.stack([v1["W2"], v2["W2"]])
    a2 = jnp.stack([jnp.stack([v1["b2"], v1["bn2_g"], v1["bn2_b"]]),
                    jnp.stack([v2["b2"], v2["bn2_g"], v2["bn2_b"]])])
    w3 = jnp.stack([jnp.pad(v1["W3"], ((0, 0), (0, D - 1))),
                    jnp.pad(v2["W3"], ((0, 0), (0, D - 1)))])
    b3 = jnp.stack([jnp.broadcast_to(v1["b3"], (1, D)),
                    jnp.broadcast_to(v2["b3"], (1, D))])
    qf, gslf = _tc5(z, zsq, bn0, w1, a1, w2, a2, w3, b3)

    q1o = qf[0, :, 0]
    q2o = qf[1, :, 0]
    side = (jnp.sum(sse_out[0]) / E + jnp.sum(sse_out[1]) / E
            + gslf[0, 0, 0] + gslf[1, 0, 0]) / 4.0
    return (q1o, q2o, side)


# speedup vs baseline: 2.6198x; 1.0017x over previous
import functools

import jax
import jax.numpy as jnp
from jax import lax
from jax.experimental import pallas as pl
from jax.experimental.pallas import tpu as pltpu
from jax.experimental.pallas import tpu_sc as plsc

D = 128
N = 10000
E = 160000
E2 = 2 * E
MS = 131072
MAX_EP_LEN = 50.0

NCH2 = 2560
CPW2 = NCH2 // 16
E2PAD = NCH2 * 128
NCH1 = 1280
CPW1 = NCH1 // 16
E1PAD = NCH1 * 128
AGG_ROWS = 10240
NCHS = MS // 128
CPWS = NCHS // 16
TROWS = 4 * N + 8
SEG_R = 8192
NPASS = MS // SEG_R
EPW5 = MS // 16

_f32 = jnp.float32
_i32 = jnp.int32


def _lrelu(x):
    return jnp.maximum(x, 0.01 * x)


def _tc1_body(x_ref, w_ref, b_ref, o_ref):
    o_ref[...] = jnp.dot(x_ref[...], w_ref[...],
                         preferred_element_type=_f32) + b_ref[...]


def _tc1(x, w, b):
    return pl.pallas_call(
        _tc1_body,
        grid=(10,),
        in_specs=[
            pl.BlockSpec((1000, D), lambda i: (i, 0)),
            pl.BlockSpec((D, 512), lambda i: (0, 0)),
            pl.BlockSpec((1, 512), lambda i: (0, 0)),
        ],
        out_specs=pl.BlockSpec((1000, 512), lambda i: (i, 0)),
        out_shape=jax.ShapeDtypeStruct((N, 512), _f32),
    )(x, w, b)


def _tc2_body(x_ref, agg_ref, u1_ref, u2_ref, bu_ref, we_ref, be_ref, o_ref):
    x = x_ref[...]
    outs = []
    for q in range(2):
        h = _lrelu(jnp.dot(x, u1_ref[q], preferred_element_type=_f32)
                   + jnp.dot(agg_ref[q], u2_ref[q], preferred_element_type=_f32)
                   + bu_ref[q])
        outs.append(jnp.dot(h, we_ref[q], preferred_element_type=_f32) + be_ref[q])
    o_ref[...] = jnp.concatenate(outs, axis=1)


def _tc2(x, agg, u1, u2, bu, we, be):
    return pl.pallas_call(
        _tc2_body,
        grid=(10,),
        in_specs=[
            pl.BlockSpec((1000, D), lambda i: (i, 0)),
            pl.BlockSpec((2, 1000, D), lambda i: (0, i, 0)),
            pl.BlockSpec((2, D, D), lambda i: (0, 0, 0)),
            pl.BlockSpec((2, D, D), lambda i: (0, 0, 0)),
            pl.BlockSpec((2, 1, D), lambda i: (0, 0, 0)),
            pl.BlockSpec((2, D, 2 * D), lambda i: (0, 0, 0)),
            pl.BlockSpec((2, 1, 2 * D), lambda i: (0, 0, 0)),
        ],
        out_specs=pl.BlockSpec((1000, 512), lambda i: (i, 0)),
        out_shape=jax.ShapeDtypeStruct((N, 512), _f32),
    )(x, agg, u1, u2, bu, we, be)


def _tc3_body(sub_ref, w_ref, b_ref, o_ref):
    o_ref[...] = jnp.dot(sub_ref[...], w_ref[0],
                         preferred_element_type=_f32) + b_ref[0]


def _tc3(sub, w, b):
    return pl.pallas_call(
        _tc3_body,
        grid=(2, 64),
        in_specs=[
            pl.BlockSpec((2048, D), lambda q, i: (q * 64 + i, 0)),
            pl.BlockSpec((1, D, 384), lambda q, i: (q, 0, 0)),
            pl.BlockSpec((1, 1, 384), lambda q, i: (q, 0, 0)),
        ],
        out_specs=pl.BlockSpec((2048, 384), lambda q, i: (q * 64 + i, 0)),
        out_shape=jax.ShapeDtypeStruct((2 * MS, 384), _f32),
    )(sub, w, b)


def _tc4_body(g_ref, agg2_ref, w_ref, z_ref, zsq_ref):
    sub_u1 = g_ref[...][:, 256:384]
    mm = jnp.dot(agg2_ref[0], w_ref[0], preferred_element_type=_f32)
    h = _lrelu(sub_u1 + mm)
    hsq = h * h
    for g in range(16):
        z_ref[0, g, :] = jnp.mean(h[g * 128:(g + 1) * 128], axis=0)
        zsq_ref[0, g, :] = jnp.mean(hsq[g * 128:(g + 1) * 128], axis=0)


def _tc4(ginter, agg2, w):
    return pl.pallas_call(
        _tc4_body,
        grid=(2, 64),
        in_specs=[
            pl.BlockSpec((2048, 384), lambda q, i: (q * 64 + i, 0)),
            pl.BlockSpec((1, 2048, D), lambda q, i: (q, i, 0)),
            pl.BlockSpec((1, D, D), lambda q, i: (q, 0, 0)),
        ],
        out_specs=[
            pl.BlockSpec((1, 16, D), lambda q, i: (q, i, 0)),
            pl.BlockSpec((1, 16, D), lambda q, i: (q, i, 0)),
        ],
        out_shape=[
            jax.ShapeDtypeStruct((2, 1024, D), _f32),
            jax.ShapeDtypeStruct((2, 1024, D), _f32),
        ],
    )(ginter, agg2, w)


def _tc5_body(z_ref, zsq_ref, bn0_ref, w1_ref, a1_ref, w2_ref, a2_ref,
              w3_ref, b3_ref, q_ref, gsl_ref):
    z = z_ref[0]
    gsl = jnp.mean(zsq_ref[0])

    def bn_lrelu(a, g, b):
        mu = jnp.mean(a, axis=0)
        var = jnp.mean(a * a, axis=0) - mu * mu
        return _lrelu((a - mu) * lax.rsqrt(var + 1e-5) * g + b)

    z0 = bn_lrelu(z, bn0_ref[0, 0], bn0_ref[0, 1])
    a1 = jnp.dot(z0, w1_ref[0], preferred_element_type=_f32) + a1_ref[0, 0]
    a1 = bn_lrelu(a1, a1_ref[0, 1], a1_ref[0, 2])
    a2 = jnp.dot(a1, w2_ref[0], preferred_element_type=_f32) + a2_ref[0, 0]
    a2 = bn_lrelu(a2, a2_ref[0, 1], a2_ref[0, 2])
    q_ref[0] = jnp.dot(a2, w3_ref[0], preferred_element_type=_f32) + b3_ref[0]
    gsl_ref[0] = jnp.full((8, D), gsl, _f32)


def _tc5(z, zsq, bn0, w1, a1, w2, a2, w3, b3):
    return pl.pallas_call(
        _tc5_body,
        grid=(2,),
        in_specs=[
            pl.BlockSpec((1, 1024, D), lambda q: (q, 0, 0)),
            pl.BlockSpec((1, 1024, D), lambda q: (q, 0, 0)),
            pl.BlockSpec((1, 2, D), lambda q: (q, 0, 0)),
            pl.BlockSpec((1, D, 256), lambda q: (q, 0, 0)),
            pl.BlockSpec((1, 3, 256), lambda q: (q, 0, 0)),
            pl.BlockSpec((1, 256, 256), lambda q: (q, 0, 0)),
            pl.BlockSpec((1, 3, 256), lambda q: (q, 0, 0)),
            pl.BlockSpec((1, 256, D), lambda q: (q, 0, 0)),
            pl.BlockSpec((1, 1, D), lambda q: (q, 0, 0)),
        ],
        out_specs=[
            pl.BlockSpec((1, 1024, D), lambda q: (q, 0, 0)),
            pl.BlockSpec((1, 8, D), lambda q: (q, 0, 0)),
        ],
        out_shape=[
            jax.ShapeDtypeStruct((2, 1024, D), _f32),
            jax.ShapeDtypeStruct((2, 8, D), _f32),
        ],
    )(z, zsq, bn0, w1, a1, w2, a2, w3, b3)


def _sc_mesh():
    return plsc.VectorSubcoreMesh(core_axis_name="c", subcore_axis_name="s")


CH2 = 64
GRP2R = 16


def _sc_msg_call(t_tab, srcidx, dstidx, scatidx, zer):
    @functools.partial(
        pl.kernel,
        mesh=_sc_mesh(),
        out_type=jax.ShapeDtypeStruct((2, AGG_ROWS, D), _f32),
        scratch_types=[
            pltpu.VMEM((GRP2R, 128), _i32),
            pltpu.VMEM((GRP2R, 128), _i32),
            pltpu.VMEM((GRP2R, 128), _i32),
            pltpu.VMEM((CH2,), _i32),
            pltpu.VMEM((CH2, D), _f32),
            pltpu.VMEM((CH2, D), _f32),
            pltpu.VMEM((CH2, D), _f32),
            pltpu.VMEM((CH2, D), _f32),
            pltpu.SemaphoreType.DMA,
            pltpu.SemaphoreType.DMA,
            pltpu.VMEM_SHARED((AGG_ROWS, D), _f32),
        ],
    )
    def body(t_hbm, src_hbm, dst_hbm, sc_hbm, zer_hbm, out_hbm,
             sidx, didx, scat, scbuf, xs0, xd0, xs1, xd1, sem0, sem1, aggsh):
        c = lax.axis_index("c")
        s = lax.axis_index("s")
        for t in range(5):
            pltpu.sync_copy(zer_hbm.at[pl.ds(0, 128)],
                            aggsh.at[pl.ds(s * 640 + t * 128, 128)])
        plsc.subcore_barrier()

        def issue(r, h, xsb, xdb, sem):
            pltpu.async_copy(t_hbm.at[sidx.at[r, pl.ds(h, CH2)]], xsb, sem)
            pltpu.async_copy(t_hbm.at[didx.at[r, pl.ds(h, CH2)]], xdb, sem)

        def wait2(xsb, xdb, sem):
            pltpu.make_async_copy(t_hbm.at[sidx.at[0, pl.ds(0, CH2)]], xsb,
                                  sem).wait()
            pltpu.make_async_copy(t_hbm.at[didx.at[0, pl.ds(0, CH2)]], xdb,
                                  sem).wait()

        def process(r, h, xsb, xdb):
            @plsc.parallel_loop(0, CH2, 1, unroll=4)
            def _row(rr):
                for b in range(8):
                    sl = pl.ds(b * 16, 16)
                    t = xsb[rr, sl] + xdb[rr, sl]
                    xsb[rr, sl] = jnp.maximum(t, 0.01 * t)
            for k in range(CH2 // 16):
                scbuf[pl.ds(k * 16, 16)] = scat[r, pl.ds(h + k * 16, 16)]
            pltpu.sync_copy(xsb, aggsh.at[scbuf], add=True)

        def group(gg, _):
            base = s * (GRP2R * 10) + gg * GRP2R
            pltpu.sync_copy(src_hbm.at[c, pl.ds(base, GRP2R)], sidx)
            pltpu.sync_copy(dst_hbm.at[c, pl.ds(base, GRP2R)], didx)
            pltpu.sync_copy(sc_hbm.at[pl.ds(base, GRP2R)], scat)
            issue(0, 0, xs0, xd0, sem0)
            issue(0, CH2, xs1, xd1, sem1)

            def pair(t, _):
                tn = (t + 1) & (GRP2R - 1)
                wait2(xs0, xd0, sem0)
                process(t, 0, xs0, xd0)
                issue(tn, 0, xs0, xd0, sem0)
                wait2(xs1, xd1, sem1)
                process(t, CH2, xs1, xd1)
                issue(tn, CH2, xs1, xd1, sem1)
                return 0

            lax.fori_loop(0, GRP2R, pair, 0)
            wait2(xs0, xd0, sem0)
            wait2(xs1, xd1, sem1)
            return 0

        lax.fori_loop(0, 10, group, 0)
        plsc.subcore_barrier()
        pltpu.sync_copy(aggsh.at[pl.ds(s * 640, 640)],
                        out_hbm.at[c, pl.ds(s * 640, 640)])

    return body(t_tab, srcidx, dstidx, scatidx, zer)


def _sc_edge_call(h_tab, srcidx, dstidx, ang, act, gt, wv, subidx):
    @functools.partial(
        pl.kernel,
        mesh=_sc_mesh(),
        out_type=(
            jax.ShapeDtypeStruct((2 * E1PAD, D), _f32),
            jax.ShapeDtypeStruct((2, 128, 16), _f32),
            jax.ShapeDtypeStruct((2 * MS, D), _f32),
        ),
        scratch_types=[
            pltpu.VMEM((GRP2R, 128), _i32),
            pltpu.VMEM((GRP2R, 128), _i32),
            pltpu.VMEM((GRP2R, 128), _f32),
            pltpu.VMEM((GRP2R, 128), _f32),
            pltpu.VMEM((GRP2R, 128), _f32),
            pltpu.VMEM((3, D), _f32),
            pltpu.VMEM((CPWS, 128), _i32),
            pltpu.VMEM((CH2, D), _f32),
            pltpu.VMEM((CH2, D), _f32),
            pltpu.VMEM((CH2, D), _f32),
            pltpu.VMEM((CH2, D), _f32),
            pltpu.VMEM((8, 16), _f32),
            pltpu.SemaphoreType.DMA,
            pltpu.SemaphoreType.DMA,
        ],
    )
    def body(h_hbm, src_hbm, dst_hbm, ang_hbm, act_hbm, gt_hbm, wv_hbm,
             sub_hbm, ef_hbm, sse_hbm, subo_hbm,
             sidx, didx, angv, actv, gtv, wvv, subi, hs0, hd0, hs1, hd1,
             ssev, sem0, sem1):
        c = lax.axis_index("c")
        s = lax.axis_index("s")
        pltpu.sync_copy(wv_hbm.at[c], wvv)
        pltpu.sync_copy(sub_hbm.at[c, pl.ds(s * CPWS, CPWS)], subi)

        wa = [wvv[0, pl.ds(b * 16, 16)] for b in range(8)]
        wb = [wvv[1, pl.ds(b * 16, 16)] for b in range(8)]
        ws = [wvv[2, pl.ds(b * 16, 16)] for b in range(8)]
        lanes = lax.iota(_i32, 16)
        perms = [lanes ^ k for k in (8, 4, 2, 1)]
        efbase = (c * NCH1 + s * CPW1) * 128

        def issue(r, h, hsb, hdb, sem):
            pltpu.async_copy(h_hbm.at[sidx.at[r, pl.ds(h, CH2)]], hsb, sem)
            pltpu.async_copy(h_hbm.at[didx.at[r, pl.ds(h, CH2)]], hdb, sem)

        def wait2(hsb, hdb, sem):
            pltpu.make_async_copy(h_hbm.at[sidx.at[0, pl.ds(0, CH2)]], hsb,
                                  sem).wait()
            pltpu.make_async_copy(h_hbm.at[didx.at[0, pl.ds(0, CH2)]], hdb,
                                  sem).wait()

        def process(gg, r, h, hsb, hdb, sse_in):
            def grp(g, sse_g):
                svec = jnp.zeros((16,), _f32)
                av = angv[r, pl.ds(h + g * 16, 16)]
                cv = actv[r, pl.ds(h + g * 16, 16)]
                for r16 in range(16):
                    rr = g * 16 + r16
                    a_s = av[r16]
                    c_s = cv[r16]
                    p = None
                    for b in range(8):
                        sl = pl.ds(b * 16, 16)
                        t = (hsb[rr, sl] + hdb[rr, sl]
                             + a_s * wa[b] + c_s * wb[b])
                        e = jnp.maximum(t, 0.01 * t)
                        hsb[rr, sl] = e
                        p = e * ws[b] if p is None else p + e * ws[b]
                    for pm in perms:
                        p = p + p.at[pm].get(mode="promise_in_bounds")
                    svec = jnp.where(lanes == r16, p, svec)
                sig = 1.0 / (1.0 + jnp.exp(-svec))
                dlt = sig - gtv[r, pl.ds(h + g * 16, 16)]
                return sse_g + dlt * dlt

            sse_o = lax.fori_loop(0, CH2 // 16, grp, sse_in)
            pltpu.sync_copy(
                hsb, ef_hbm.at[pl.ds(efbase + (gg * GRP2R + r) * 128 + h,
                                     CH2)])
            return sse_o

        def group(gg, sse_in):
            base = s * CPW1 + gg * GRP2R
            pltpu.sync_copy(src_hbm.at[c, pl.ds(base, GRP2R)], sidx)
            pltpu.sync_copy(dst_hbm.at[c, pl.ds(base, GRP2R)], didx)
            pltpu.sync_copy(ang_hbm.at[pl.ds(base, GRP2R)], angv)
            pltpu.sync_copy(act_hbm.at[pl.ds(base, GRP2R)], actv)
            pltpu.sync_copy(gt_hbm.at[pl.ds(base, GRP2R)], gtv)
            issue(0, 0, hs0, hd0, sem0)
            issue(0, CH2, hs1, hd1, sem1)

            def pair(t, sse_p):
                tn = (t + 1) & (GRP2R - 1)
                wait2(hs0, hd0, sem0)
                sse_p = process(gg, t, 0, hs0, hd0, sse_p)
                issue(tn, 0, hs0, hd0, sem0)
                wait2(hs1, hd1, sem1)
                sse_p = process(gg, t, CH2, hs1, hd1, sse_p)
                issue(tn, CH2, hs1, hd1, sem1)
                return sse_p

            sse_o = lax.fori_loop(0, GRP2R, pair, sse_in)
            wait2(hs0, hd0, sem0)
            wait2(hs1, hd1, sem1)
            return sse_o

        sse = lax.fori_loop(0, CPW1 // GRP2R, group, jnp.zeros((16,), _f32))
        ssev[0, :] = sse
        for t in range(1, 8):
            ssev[t, :] = jnp.zeros((16,), _f32)
        pltpu.sync_copy(ssev, sse_hbm.at[c, pl.ds(s * 8, 8)])
        plsc.subcore_barrier()

        subbase = (c * NCHS + s * CPWS) * 128

        def sgissue(k, hsb, sem):
            pltpu.async_copy(ef_hbm.at[subi.at[k >> 1, pl.ds((k & 1) * CH2,
                                                             CH2)]],
                             hsb, sem)

        def sgwait(hsb, sem):
            pltpu.make_async_copy(ef_hbm.at[subi.at[0, pl.ds(0, CH2)]], hsb,
                                  sem).wait()

        sgissue(0, hs0, sem0)
        sgissue(1, hs1, sem1)
        nsg = 2 * CPWS

        def sg(t, _):
            k0 = t * 2
            sgwait(hs0, sem0)
            pltpu.sync_copy(hs0, subo_hbm.at[pl.ds(subbase + k0 * CH2, CH2)])
            sgissue((k0 + 2) & (nsg - 1), hs0, sem0)
            sgwait(hs1, sem1)
            pltpu.sync_copy(hs1, subo_hbm.at[pl.ds(subbase + (k0 + 1) * CH2,
                                                   CH2)])
            sgissue((k0 + 3) & (nsg - 1), hs1, sem1)
            return 0

        lax.fori_loop(0, CPWS, sg, 0)
        sgwait(hs0, sem0)
        sgwait(hs1, sem1)

    return body(h_tab, srcidx, dstidx, ang, act, gt, wv, subidx)


WCAP = 8320


def _sc_sub_call(g_tab, gs, gd, zer):
    @functools.partial(
        pl.kernel,
        mesh=_sc_mesh(),
        out_type=jax.ShapeDtypeStruct((2, MS, D), _f32),
        scratch_types=[
            pltpu.VMEM((EPW5,), _i32),
            pltpu.VMEM((EPW5,), _i32),
            pltpu.VMEM((2, 128), _i32),
            pltpu.VMEM((2, 128), _i32),
            pltpu.VMEM((2, 128), _i32),
            pltpu.VMEM((2, 128), _i32),
            pltpu.VMEM((64,), _i32),
            pltpu.VMEM((64,), _i32),
            pltpu.VMEM((64,), _i32),
            pltpu.VMEM((64, D), _f32),
            pltpu.VMEM((64, D), _f32),
            pltpu.VMEM((48,), _i32),
            pltpu.VMEM_SHARED((16 * WCAP,), _i32),
            pltpu.VMEM_SHARED((16 * WCAP,), _i32),
            pltpu.VMEM_SHARED((16 * WCAP,), _i32),
            pltpu.VMEM_SHARED((SEG_R + 128, D), _f32),
            pltpu.SemaphoreType.DMA,
            pltpu.SemaphoreType.DMA,
            pltpu.SemaphoreType.DMA,
            pltpu.SemaphoreType.DMA,
        ],
    )
    def body(g_hbm, gs_hbm, gd_hbm, zer_hbm, out_hbm,
             gsv, gdv, tgtstg, vals, vald, valr, cidxs, cidxd, cidxr,
             arows, brows, cntv, bkts, bktd, bktr, spm, sem1, sem2,
             semc0, semc1):
        c = lax.axis_index("c")
        s = lax.axis_index("s")
        cq = c * (3 * MS)
        wbase = s * WCAP
        trash = wbase + WCAP - 1
        lanes = lax.iota(_i32, 16)
        pltpu.sync_copy(gs_hbm.at[pl.ds(s * EPW5, EPW5)], gsv)
        pltpu.sync_copy(gd_hbm.at[pl.ds(s * EPW5, EPW5)], gdv)
        for t in range(4):
            pltpu.sync_copy(zer_hbm.at[pl.ds(0, 128)],
                            spm.at[pl.ds(s * 520 + t * 128, 128)])
        pltpu.sync_copy(zer_hbm.at[pl.ds(0, 8)],
                        spm.at[pl.ds(s * 520 + 512, 8)])
        plsc.subcore_barrier()

        kmask = {k: jnp.minimum(jnp.maximum(lanes - (k - 1), 0), 1)
                 for k in (1, 2, 4, 8)}
        trash16 = jnp.zeros((16,), _i32) + trash

        def cwait(stg, sem):
            pltpu.make_async_copy(stg, bkts.at[tgtstg.at[0]], sem).wait()
            pltpu.make_async_copy(stg, bktd.at[tgtstg.at[0]], sem).wait()
            pltpu.make_async_copy(stg, bktr.at[tgtstg.at[0]], sem).wait()

        def pass_body(p, _):
            lo = p * SEG_R

            for par in range(2):
                for k in range(8):
                    tgtstg[par, pl.ds(k * 16, 16)] = trash16
            for par, semc in ((0, semc0), (1, semc1)):
                pltpu.async_copy(vals.at[par], bkts.at[tgtstg.at[par]], semc)
                pltpu.async_copy(vald.at[par], bktd.at[tgtstg.at[par]], semc)
                pltpu.async_copy(valr.at[par], bktr.at[tgtstg.at[par]], semc)

            def dbatch(tt, off_in):
                off_b = off_in
                for par, semc in ((0, semc0), (1, semc1)):
                    bb = tt * 2 + par
                    cwait(vals.at[par], semc)
                    for gg in range(8):
                        sl = pl.ds(bb * 128 + gg * 16, 16)
                        gsvv = gsv[sl]
                        gdvv = gdv[sl]
                        rel = gdvv - lo
                        ind = ((rel >> 31) + 1) * ((rel - SEG_R) >> 31) * (-1)
                        pf = ind
                        for k in (1, 2, 4, 8):
                            sh = pf.at[jnp.maximum(lanes - k, 0)].get(
                                mode="promise_in_bounds")
                            pf = pf + sh * kmask[k]
                        osl = pl.ds(gg * 16, 16)
                        tgtstg[par, osl] = trash + (wbase + off_b + pf - 1
                                                    - trash) * ind
                        vals[par, osl] = 3 * gsvv + cq
                        vald[par, osl] = 3 * gdvv + (cq + 1)
                        valr[par, osl] = rel
                        off_b = off_b + pf[15]
                    pltpu.async_copy(vals.at[par], bkts.at[tgtstg.at[par]],
                                     semc)
                    pltpu.async_copy(vald.at[par], bktd.at[tgtstg.at[par]],
                                     semc)
                    pltpu.async_copy(valr.at[par], bktr.at[tgtstg.at[par]],
                                     semc)
                return off_b

            off = lax.fori_loop(0, EPW5 // 256, dbatch, jnp.int32(0))
            cwait(vals.at[0], semc0)
            cwait(vals.at[1], semc1)

            for k in range(4):
                osl = pl.ds(k * 16, 16)
                tgtstg[0, osl] = wbase + off + k * 16 + lanes
                vals[0, osl] = jnp.zeros((16,), _i32)
                vald[0, osl] = jnp.zeros((16,), _i32)
                valr[0, osl] = jnp.full((16,), SEG_R, _i32)
            for k in range(4, 8):
                tgtstg[0, pl.ds(k * 16, 16)] = trash16
            pltpu.sync_copy(vals.at[0], bkts.at[tgtstg.at[0]])
            pltpu.sync_copy(vald.at[0], bktd.at[tgtstg.at[0]])
            pltpu.sync_copy(valr.at[0], bktr.at[tgtstg.at[0]])
            nch = (off + 63) // 64

            def surv(j, _):
                cb = pl.multiple_of(wbase + j * 64, 64)
                pltpu.async_copy(bkts.at[pl.ds(cb, 64)], cidxs, semc0)
                pltpu.async_copy(bktd.at[pl.ds(cb, 64)], cidxd, semc0)
                pltpu.async_copy(bktr.at[pl.ds(cb, 64)], cidxr, semc0)
                pltpu.make_async_copy(bkts.at[pl.ds(0, 64)], cidxs,
                                      semc0).wait()
                pltpu.make_async_copy(bkts.at[pl.ds(0, 64)], cidxd,
                                      semc0).wait()
                pltpu.make_async_copy(bkts.at[pl.ds(0, 64)], cidxr,
                                      semc0).wait()
                cp1 = pltpu.async_copy(g_hbm.at[cidxs], arows, sem1)
                cp2 = pltpu.async_copy(g_hbm.at[cidxd], brows, sem2)
                cp1.wait()
                cp2.wait()

                @plsc.parallel_loop(0, 64, 1, unroll=4)
                def _row(r):
                    for b in range(8):
                        sl = pl.ds(b * 16, 16)
                        t = arows[r, sl] + brows[r, sl]
                        arows[r, sl] = jnp.maximum(t, 0.01 * t)

                pltpu.sync_copy(arows, spm.at[cidxr], add=True)
                return 0

            lax.fori_loop(0, nch, surv, 0)
            plsc.subcore_barrier()
            pltpu.sync_copy(spm.at[pl.ds(s * 512, 512)],
                            out_hbm.at[c, pl.ds(lo + s * 512, 512)])
            for t in range(4):
                pltpu.sync_copy(zer_hbm.at[pl.ds(0, 128)],
                                spm.at[pl.ds(s * 512 + t * 128, 128)])

            @pl.when(s == 15)
            def _():
                pltpu.sync_copy(zer_hbm.at[pl.ds(0, 128)],
                                spm.at[pl.ds(SEG_R, 128)])

            plsc.subcore_barrier()
            return 0

        lax.fori_loop(0, NPASS, pass_body, 0)

    return body(g_tab, gs, gd, zer)


def _pad_i32(a, n, val):
    return jnp.concatenate([a.astype(_i32), jnp.full((n - a.shape[0],), val, _i32)])


def _pad_f32(a, n, val):
    return jnp.concatenate([a.astype(_f32), jnp.full((n - a.shape[0],), val, _f32)])


def kernel(node_features, sup_masses, actions, angles, gt_edges, round_n,
           params, edge_index, sub_graphs_0, sep_subgraphs_0):
    p1, p2 = params["q1"], params["q2"]
    g1, g2 = params["g1"], params["g2"]
    v1, v2 = params["v1"], params["v2"]

    rn = round_n / MAX_EP_LEN
    x = jnp.concatenate(
        [node_features, sup_masses, jnp.ones_like(sup_masses) * rn], axis=1)

    s0 = edge_index[0].astype(_i32)
    d0 = edge_index[1].astype(_i32)
    e2src = jnp.concatenate([s0, d0])
    e2dst = jnp.concatenate([d0, s0])

    srcabs2 = jnp.stack([_pad_i32(4 * e2src + 2 * q, E2PAD, 4 * N)
                         for q in range(2)]).reshape(2, NCH2, 128)
    dstabs2 = jnp.stack([_pad_i32(4 * e2dst + 2 * q + 1, E2PAD, 4 * N)
                         for q in range(2)]).reshape(2, NCH2, 128)
    scat2 = _pad_i32(e2dst, E2PAD, N).reshape(NCH2, 128)

    srcabs1 = jnp.stack([_pad_i32(4 * s0 + 2 * q, E1PAD, 4 * N)
                         for q in range(2)]).reshape(2, NCH1, 128)
    dstabs1 = jnp.stack([_pad_i32(4 * d0 + 2 * q + 1, E1PAD, 4 * N)
                         for q in range(2)]).reshape(2, NCH1, 128)
    angp = _pad_f32(angles, E1PAD, 0.0).reshape(NCH1, 128)
    actp = _pad_f32(actions, E1PAD, 0.0).reshape(NCH1, 128)
    gtp = _pad_f32(gt_edges, E1PAD, 0.5).reshape(NCH1, 128)

    subidx = jnp.stack([sub_graphs_0.astype(_i32) + q * E1PAD
                        for q in range(2)]).reshape(2, NCHS, 128)

    gs = jnp.concatenate([sep_subgraphs_0[0], sep_subgraphs_0[1]]).astype(_i32)
    gd = jnp.concatenate([sep_subgraphs_0[1], sep_subgraphs_0[0]]).astype(_i32)

    zer = jnp.zeros((128, D), _f32)

    wcat = jnp.concatenate([p1["W_msg"][:D], p1["W_msg"][D:],
                            p2["W_msg"][:D], p2["W_msg"][D:]], axis=1)
    bcat = jnp.concatenate([jnp.zeros((D,), _f32), p1["b_msg"],
                            jnp.zeros((D,), _f32), p2["b_msg"]])[None, :]
    t_tab = jnp.concatenate([_tc1(x, wcat, bcat).reshape(4 * N, D),
                             jnp.zeros((8, D), _f32)], axis=0)

    agg = _sc_msg_call(t_tab, srcabs2, dstabs2, scat2, zer)[:, :N]

    u1 = jnp.stack([p1["W_upd"][:D], p2["W_upd"][:D]])
    u2 = jnp.stack([p1["W_upd"][D:], p2["W_upd"][D:]])
    bu = jnp.stack([p1["b_upd"], p2["b_upd"]])[:, None, :]
    we = jnp.stack([jnp.concatenate([p1["W_edge"][:D], p1["W_edge"][D:2 * D]], axis=1),
                    jnp.concatenate([p2["W_edge"][:D], p2["W_edge"][D:2 * D]], axis=1)])
    be = jnp.stack([jnp.concatenate([jnp.zeros((D,), _f32), p1["b_edge"]]),
                    jnp.concatenate([jnp.zeros((D,), _f32), p2["b_edge"]])])[:, None, :]
    h_tab = jnp.concatenate([_tc2(x, agg, u1, u2, bu, we, be).reshape(4 * N, D),
                             jnp.zeros((8, D), _f32)], axis=0)

    wv = jnp.stack([jnp.stack([p1["W_edge"][2 * D], p1["W_edge"][2 * D + 1], p1["w_side"]]),
                    jnp.stack([p2["W_edge"][2 * D], p2["W_edge"][2 * D + 1], p2["w_side"]])])
    ef, sse_out, sub = _sc_edge_call(h_tab, srcabs1, dstabs1, angp, actp, gtp,
                                     wv, subidx)

    wg = jnp.stack([jnp.concatenate([g1["W_msg"][:D], g1["W_msg"][D:], g1["W_upd"][:D]], axis=1),
                    jnp.concatenate([g2["W_msg"][:D], g2["W_msg"][D:], g2["W_upd"][:D]], axis=1)])
    bg = jnp.stack([jnp.concatenate([jnp.zeros((D,), _f32), g1["b_msg"], g1["b_upd"]]),
                    jnp.concatenate([jnp.zeros((D,), _f32), g2["b_msg"], g2["b_upd"]])])[:, None, :]
    ginter = _tc3(sub, wg, bg)
    g_tab = ginter.reshape(6 * MS, D)

    agg2 = _sc_sub_call(g_tab, gs, gd, zer)

    wu2 = jnp.stack([g1["W_upd"][D:], g2["W_upd"][D:]])
    z, zsq = _tc4(ginter, agg2, wu2)

    bn0 = jnp.stack([jnp.stack([v1["bn0_g"], v1["bn0_b"]]),
                     jnp.stack([v2["bn0_g"], v2["bn0_b"]])])
    w1 = jnp.stack([v1["W1"], v2["W1"]])
    a1 = jnp.stack([jnp.stack([v1["b1"], v1["bn1_g"], v1["bn1_b"]]),
                    jnp.stack([v2["b1"], v2["bn1_g"], v2["bn1_b"]])])
    w2 = jnp.stack([v1["W2"], v2["W2"]])
    a2 = jnp.stack([jnp.stack([v1["b2"], v1["bn2_g"], v1["bn2_b"]]),
                    jnp.stack([v2["b2"], v2["bn2_g"], v2["bn2_b"]])])
    w3 = jnp.stack([jnp.pad(v1["W3"], ((0, 0), (0, D - 1))),
                    jnp.pad(v2["W3"], ((0, 0), (0, D - 1)))])
    b3 = jnp.stack([jnp.broadcast_to(v1["b3"], (1, D)),
                    jnp.broadcast_to(v2["b3"], (1, D))])
    qf, gslf = _tc5(z, zsq, bn0, w1, a1, w2, a2, w3, b3)

    q1o = qf[0, :, 0]
    q2o = qf[1, :, 0]
    side = (jnp.sum(sse_out[0]) / E + jnp.sum(sse_out[1]) / E
            + gslf[0, 0, 0] + gslf[1, 0, 0]) / 4.0
    return (q1o, q2o, side)
